# baseline (speedup 1.0000x reference)
import functools

import jax
import jax.numpy as jnp
from jax import lax
from jax.experimental import pallas as pl
from jax.experimental.pallas import tpu as pltpu
from jax.experimental.pallas import tpu_sc as plsc

NC = 2
NS = 16
NW = NC * NS
B = 96
LANES = 16
CW = 128


def _sc_segment_sum(nb, np_pad, h):
  ZB = 64
  n_chunks = np_pad // ZB
  q = -(-np_pad // CW)
  np_cpad = -(-q // 8) * 8
  cnt_rpt = 8
  cnt_nchunks = np_cpad // cnt_rpt

  mesh = plsc.VectorSubcoreMesh(core_axis_name="c", subcore_axis_name="s",
                                num_cores=NC, num_subcores=NS)

  @functools.partial(
      pl.kernel,
      out_type=(
          jax.ShapeDtypeStruct((NC, np_pad, h), jnp.float32),
          jax.ShapeDtypeStruct((NC, np_cpad, CW), jnp.float32),
      ),
      mesh=mesh,
      scratch_types=[
          pltpu.VMEM((2, B), jnp.int32),
          pltpu.VMEM((2, B), jnp.int32),
          pltpu.VMEM((B,), jnp.int32),
          pltpu.VMEM((B,), jnp.int32),
          pltpu.VMEM((2, B, h), jnp.float32),
          pltpu.VMEM_SHARED((np_pad, h), jnp.float32),
          pltpu.VMEM_SHARED((np_cpad, CW), jnp.float32),
          pltpu.SemaphoreType.DMA,
          pltpu.SemaphoreType.DMA,
      ],
  )
  def k(xc_hbm, oh_hbm, src_hbm, dst_hbm, acc_out, cnt_out,
        src_v, dst_v, hi_v, lo_v, rows_v, acc_sh, cnt_sh, g0, g1):
    c = lax.axis_index("c")
    s = lax.axis_index("s")
    wid = s * NC + c
    gsem = (g0, g1)

    def init_row(i, _):
      for q in range(h // LANES):
        rows_v[0, i, pl.ds(q * LANES, LANES)] = jnp.zeros((LANES,), jnp.float32)
      return _
    lax.fori_loop(0, B, init_row, None)

    @pl.loop(s, n_chunks, step=NS)
    def zero_chunk(kk):
      pltpu.sync_copy(rows_v.at[0, pl.ds(0, ZB)], acc_sh.at[pl.ds(kk * ZB, ZB)])

    @pl.loop(s, cnt_nchunks, step=NS)
    def zero_cnt(kk):
      pltpu.sync_copy(rows_v.at[0, pl.ds(0, cnt_rpt)],
                      cnt_sh.at[pl.ds(kk * cnt_rpt, cnt_rpt)])
    plsc.subcore_barrier()

    def split_src(b):
      for q in range(B // LANES):
        s16 = src_v[b, pl.ds(q * LANES, LANES)]
        hi_v[pl.ds(q * LANES, LANES)] = lax.shift_right_logical(s16, 7)
        lo_v[pl.ds(q * LANES, LANES)] = lax.bitwise_and(s16, CW - 1)

    def load_idx(j, b):
      pltpu.sync_copy(src_hbm.at[wid, j], src_v.at[b])
      pltpu.sync_copy(dst_hbm.at[wid, j], dst_v.at[b])

    load_idx(0, 0)
    pltpu.async_copy(xc_hbm.at[dst_v.at[0]], rows_v.at[0], gsem[0])

    @pl.loop(0, nb // 2)
    def lp(j2):
      for b in (0, 1):
        b1 = 1 - b
        j = j2 * 2 + b

        @pl.when(j + 1 < nb)
        def _():
          load_idx(j + 1, b1)
          pltpu.async_copy(xc_hbm.at[dst_v.at[b1]], rows_v.at[b1], gsem[b1])

        split_src(b)
        pltpu.make_async_copy(xc_hbm.at[dst_v.at[b]], rows_v.at[b], gsem[b]).wait()
        pltpu.sync_copy(rows_v.at[b], acc_sh.at[src_v.at[b]], add=True)
        pltpu.async_copy(oh_hbm.at[lo_v], rows_v.at[b], gsem[b]).wait()
        pltpu.sync_copy(rows_v.at[b], cnt_sh.at[hi_v], add=True)

    plsc.subcore_barrier()

    @pl.loop(s, n_chunks, step=NS)
    def out_chunk(kk):
      pltpu.sync_copy(acc_sh.at[pl.ds(kk * ZB, ZB)], rows_v.at[0, pl.ds(0, ZB)])
      pltpu.sync_copy(rows_v.at[0, pl.ds(0, ZB)], acc_out.at[c, pl.ds(kk * ZB, ZB)])

    @pl.loop(s, cnt_nchunks, step=NS)
    def out_cnt(kk):
      pltpu.sync_copy(cnt_sh.at[pl.ds(kk * cnt_rpt, cnt_rpt)], rows_v.at[1, pl.ds(0, cnt_rpt)])
      pltpu.sync_copy(rows_v.at[1, pl.ds(0, cnt_rpt)], cnt_out.at[c, pl.ds(kk * cnt_rpt, cnt_rpt)])

  return k


def _dense_body(xp_ref, acc_ref, cnt_ref, w1_ref, b1_ref, w2_ref, b2_ref,
                wu_ref, bu_ref, g_ref, bt_ref, out_ref):
  xp = xp_ref[...]
  ssum = acc_ref[0] + acc_ref[1]
  cnt = cnt_ref[0] + cnt_ref[1]
  hid = jnp.maximum(
      jnp.dot(xp, w1_ref[...], preferred_element_type=jnp.float32) + b1_ref[...],
      0.0)
  pred = jnp.dot(hid, w2_ref[...], preferred_element_type=jnp.float32) + b2_ref[...]
  agg = (ssum - cnt * pred) / jnp.maximum(cnt, 1.0)
  upd = jnp.dot(agg, wu_ref[...], preferred_element_type=jnp.float32) + bu_ref[...]
  t = xp + upd
  m = jnp.mean(t, axis=1, keepdims=True)
  v = jnp.mean((t - m) * (t - m), axis=1, keepdims=True)
  out_ref[...] = (t - m) * lax.rsqrt(v + 1e-5) * g_ref[...] + bt_ref[...]


def kernel(x_parent, x_child, edge_index, W1, b1, W2, b2, Wu, bu, gamma, beta):
  np_, h = x_parent.shape
  e = edge_index.shape[1]

  np_pad = -(-(np_ + 1) // 64) * 64
  chunk = NW * B
  e_pad = -(-e // (2 * chunk)) * (2 * chunk)
  nb = e_pad // chunk

  onehot = jnp.eye(CW, dtype=jnp.float32)

  src = edge_index[0]
  dst = edge_index[1]
  pad = e_pad - e
  if pad:
    src = jnp.concatenate([src, jnp.full((pad,), np_, jnp.int32)])
    dst = jnp.concatenate([dst, jnp.zeros((pad,), jnp.int32)])
  src3 = src.reshape(NW, nb, B)
  dst3 = dst.reshape(NW, nb, B)

  acc, cnt = _sc_segment_sum(nb, np_pad, h)(x_child, onehot, src3, dst3)
  cnt_col = cnt.reshape(NC, -1, 1)

  r = 1000 if np_ % 1000 == 0 else np_
  grid = (np_ // r,)
  new_parent = pl.pallas_call(
      _dense_body,
      grid=grid,
      in_specs=[
          pl.BlockSpec((r, h), lambda i: (i, 0)),
          pl.BlockSpec((NC, r, h), lambda i: (0, i, 0)),
          pl.BlockSpec((NC, r, 1), lambda i: (0, i, 0)),
          pl.BlockSpec((h, h), lambda i: (0, 0)),
          pl.BlockSpec((1, h), lambda i: (0, 0)),
          pl.BlockSpec((h, h), lambda i: (0, 0)),
          pl.BlockSpec((1, h), lambda i: (0, 0)),
          pl.BlockSpec((h, h), lambda i: (0, 0)),
          pl.BlockSpec((1, h), lambda i: (0, 0)),
          pl.BlockSpec((1, h), lambda i: (0, 0)),
          pl.BlockSpec((1, h), lambda i: (0, 0)),
      ],
      out_specs=pl.BlockSpec((r, h), lambda i: (i, 0)),
      out_shape=jax.ShapeDtypeStruct((np_, h), jnp.float32),
  )(x_parent, acc[:, :np_], cnt_col[:, :np_],
    W1, b1.reshape(1, h), W2, b2.reshape(1, h), Wu, bu.reshape(1, h),
    gamma.reshape(1, h), beta.reshape(1, h))

  return (new_parent, x_child)


# speedup vs baseline: 1.4402x; 1.4402x over previous
import functools

import jax
import jax.numpy as jnp
from jax import lax
from jax.experimental import pallas as pl
from jax.experimental.pallas import tpu as pltpu
from jax.experimental.pallas import tpu_sc as plsc

NC = 2
NS = 16
NW = NC * NS
B = 64
LANES = 16
CW = 128


def _sc_segment_sum(nb, np_pad, h):
  ZB = 64
  n_chunks = np_pad // ZB
  q = -(-np_pad // CW)
  np_cpad = -(-q // 8) * 8
  cnt_rpt = 8
  cnt_nchunks = np_cpad // cnt_rpt

  mesh = plsc.VectorSubcoreMesh(core_axis_name="c", subcore_axis_name="s",
                                num_cores=NC, num_subcores=NS)

  @functools.partial(
      pl.kernel,
      out_type=(
          jax.ShapeDtypeStruct((NC, np_pad, h), jnp.float32),
          jax.ShapeDtypeStruct((NC, np_cpad, CW), jnp.float32),
      ),
      mesh=mesh,
      scratch_types=[
          pltpu.VMEM((2, B), jnp.int32),
          pltpu.VMEM((2, B), jnp.int32),
          pltpu.VMEM((2, B), jnp.int32),
          pltpu.VMEM((2, B), jnp.int32),
          pltpu.VMEM((2, B, h), jnp.float32),
          pltpu.VMEM((B, CW), jnp.float32),
          pltpu.VMEM_SHARED((np_pad, h), jnp.float32),
          pltpu.VMEM_SHARED((np_cpad, CW), jnp.float32),
          pltpu.SemaphoreType.DMA,
          pltpu.SemaphoreType.DMA,
          pltpu.SemaphoreType.DMA,
      ],
  )
  def k(xc_hbm, oh_hbm, src_hbm, dst_hbm, acc_out, cnt_out,
        src_v, dst_v, hi_v, lo_v, rows_v, pay_v, acc_sh, cnt_sh, g0, g1, gp):
    c = lax.axis_index("c")
    s = lax.axis_index("s")
    wid = s * NC + c
    gsem = (g0, g1)

    def init_row(i, _):
      for q in range(h // LANES):
        rows_v[0, i, pl.ds(q * LANES, LANES)] = jnp.zeros((LANES,), jnp.float32)
      return _
    lax.fori_loop(0, B, init_row, None)

    @pl.loop(s, n_chunks, step=NS)
    def zero_chunk(kk):
      pltpu.sync_copy(rows_v.at[0, pl.ds(0, ZB)], acc_sh.at[pl.ds(kk * ZB, ZB)])

    @pl.loop(s, cnt_nchunks, step=NS)
    def zero_cnt(kk):
      pltpu.sync_copy(rows_v.at[0, pl.ds(0, cnt_rpt)],
                      cnt_sh.at[pl.ds(kk * cnt_rpt, cnt_rpt)])
    plsc.subcore_barrier()

    def split_src(b):
      for q in range(B // LANES):
        s16 = src_v[b, pl.ds(q * LANES, LANES)]
        hi_v[b, pl.ds(q * LANES, LANES)] = lax.shift_right_logical(s16, 7)
        lo_v[b, pl.ds(q * LANES, LANES)] = lax.bitwise_and(s16, CW - 1)

    def load_idx(j, b):
      pltpu.sync_copy(src_hbm.at[wid, j], src_v.at[b])
      pltpu.sync_copy(dst_hbm.at[wid, j], dst_v.at[b])

    load_idx(0, 0)
    pltpu.async_copy(xc_hbm.at[dst_v.at[0]], rows_v.at[0], gsem[0])
    split_src(0)
    pltpu.async_copy(oh_hbm.at[lo_v.at[0]], pay_v, gp)

    @pl.loop(0, nb // 2)
    def lp(j2):
      for b in (0, 1):
        b1 = 1 - b
        j = j2 * 2 + b

        @pl.when(j + 1 < nb)
        def _():
          load_idx(j + 1, b1)
          pltpu.async_copy(xc_hbm.at[dst_v.at[b1]], rows_v.at[b1], gsem[b1])
          split_src(b1)

        pltpu.make_async_copy(xc_hbm.at[dst_v.at[b]], rows_v.at[b], gsem[b]).wait()
        pltpu.sync_copy(rows_v.at[b], acc_sh.at[src_v.at[b]], add=True)
        pltpu.make_async_copy(oh_hbm.at[lo_v.at[b]], pay_v, gp).wait()
        pltpu.sync_copy(pay_v, cnt_sh.at[hi_v.at[b]], add=True)

        @pl.when(j + 1 < nb)
        def _():
          pltpu.async_copy(oh_hbm.at[lo_v.at[b1]], pay_v, gp)

    plsc.subcore_barrier()

    @pl.loop(s, n_chunks, step=NS)
    def out_chunk(kk):
      pltpu.sync_copy(acc_sh.at[pl.ds(kk * ZB, ZB)], rows_v.at[0, pl.ds(0, ZB)])
      pltpu.sync_copy(rows_v.at[0, pl.ds(0, ZB)], acc_out.at[c, pl.ds(kk * ZB, ZB)])

    @pl.loop(s, cnt_nchunks, step=NS)
    def out_cnt(kk):
      pltpu.sync_copy(cnt_sh.at[pl.ds(kk * cnt_rpt, cnt_rpt)], rows_v.at[1, pl.ds(0, cnt_rpt)])
      pltpu.sync_copy(rows_v.at[1, pl.ds(0, cnt_rpt)], cnt_out.at[c, pl.ds(kk * cnt_rpt, cnt_rpt)])

  return k


def _dense_body(xp_ref, acc_ref, cnt_ref, w1_ref, b1_ref, w2_ref, b2_ref,
                wu_ref, bu_ref, g_ref, bt_ref, out_ref):
  xp = xp_ref[...]
  ssum = acc_ref[0] + acc_ref[1]
  cnt = cnt_ref[0] + cnt_ref[1]
  hid = jnp.maximum(
      jnp.dot(xp, w1_ref[...], preferred_element_type=jnp.float32) + b1_ref[...],
      0.0)
  pred = jnp.dot(hid, w2_ref[...], preferred_element_type=jnp.float32) + b2_ref[...]
  agg = (ssum - cnt * pred) / jnp.maximum(cnt, 1.0)
  upd = jnp.dot(agg, wu_ref[...], preferred_element_type=jnp.float32) + bu_ref[...]
  t = xp + upd
  m = jnp.mean(t, axis=1, keepdims=True)
  v = jnp.mean((t - m) * (t - m), axis=1, keepdims=True)
  out_ref[...] = (t - m) * lax.rsqrt(v + 1e-5) * g_ref[...] + bt_ref[...]


def kernel(x_parent, x_child, edge_index, W1, b1, W2, b2, Wu, bu, gamma, beta):
  np_, h = x_parent.shape
  e = edge_index.shape[1]

  np_pad = -(-(np_ + 1) // 64) * 64
  chunk = NW * B
  e_pad = -(-e // (2 * chunk)) * (2 * chunk)
  nb = e_pad // chunk

  onehot = jnp.eye(CW, dtype=jnp.float32)

  src = edge_index[0]
  dst = edge_index[1]
  pad = e_pad - e
  if pad:
    src = jnp.concatenate([src, jnp.full((pad,), np_, jnp.int32)])
    dst = jnp.concatenate([dst, jnp.zeros((pad,), jnp.int32)])
  src3 = src.reshape(NW, nb, B)
  dst3 = dst.reshape(NW, nb, B)

  acc, cnt = _sc_segment_sum(nb, np_pad, h)(x_child, onehot, src3, dst3)
  cnt_col = cnt.reshape(NC, -1, 1)

  r = 1000 if np_ % 1000 == 0 else np_
  grid = (np_ // r,)
  new_parent = pl.pallas_call(
      _dense_body,
      grid=grid,
      in_specs=[
          pl.BlockSpec((r, h), lambda i: (i, 0)),
          pl.BlockSpec((NC, r, h), lambda i: (0, i, 0)),
          pl.BlockSpec((NC, r, 1), lambda i: (0, i, 0)),
          pl.BlockSpec((h, h), lambda i: (0, 0)),
          pl.BlockSpec((1, h), lambda i: (0, 0)),
          pl.BlockSpec((h, h), lambda i: (0, 0)),
          pl.BlockSpec((1, h), lambda i: (0, 0)),
          pl.BlockSpec((h, h), lambda i: (0, 0)),
          pl.BlockSpec((1, h), lambda i: (0, 0)),
          pl.BlockSpec((1, h), lambda i: (0, 0)),
          pl.BlockSpec((1, h), lambda i: (0, 0)),
      ],
      out_specs=pl.BlockSpec((r, h), lambda i: (i, 0)),
      out_shape=jax.ShapeDtypeStruct((np_, h), jnp.float32),
  )(x_parent, acc[:, :np_], cnt_col[:, :np_],
    W1, b1.reshape(1, h), W2, b2.reshape(1, h), Wu, bu.reshape(1, h),
    gamma.reshape(1, h), beta.reshape(1, h))

  return (new_parent, x_child)


# speedup vs baseline: 1.4439x; 1.0026x over previous
import functools

import jax
import jax.numpy as jnp
from jax import lax
from jax.experimental import pallas as pl
from jax.experimental.pallas import tpu as pltpu
from jax.experimental.pallas import tpu_sc as plsc

NC = 2
NS = 16
NW = NC * NS
B = 64
LANES = 16
CW = 128


def _sc_segment_sum(nb, np_pad, h):
  ZB = 64
  n_chunks = np_pad // ZB
  q = -(-np_pad // CW)
  np_cpad = -(-q // 8) * 8
  cnt_rpt = 8
  cnt_nchunks = np_cpad // cnt_rpt

  mesh = plsc.VectorSubcoreMesh(core_axis_name="c", subcore_axis_name="s",
                                num_cores=NC, num_subcores=NS)

  @functools.partial(
      pl.kernel,
      out_type=(
          jax.ShapeDtypeStruct((NC, np_pad, h), jnp.float32),
          jax.ShapeDtypeStruct((NC, np_cpad, CW), jnp.float32),
      ),
      mesh=mesh,
      scratch_types=[
          pltpu.VMEM((2, B), jnp.int32),
          pltpu.VMEM((2, B), jnp.int32),
          pltpu.VMEM((2, B), jnp.int32),
          pltpu.VMEM((2, B), jnp.int32),
          pltpu.VMEM((2, B, h), jnp.float32),
          pltpu.VMEM((B, CW), jnp.float32),
          pltpu.VMEM_SHARED((np_pad, h), jnp.float32),
          pltpu.VMEM_SHARED((np_cpad, CW), jnp.float32),
          pltpu.SemaphoreType.DMA,
          pltpu.SemaphoreType.DMA,
          pltpu.SemaphoreType.DMA,
          pltpu.SemaphoreType.DMA,
          pltpu.SemaphoreType.DMA,
      ],
  )
  def k(xc_hbm, oh_hbm, src_hbm, dst_hbm, acc_out, cnt_out,
        src_v, dst_v, hi_v, lo_v, rows_v, pay_v, acc_sh, cnt_sh,
        g0, g1, gp, s0, s1):
    c = lax.axis_index("c")
    s = lax.axis_index("s")
    wid = s * NC + c
    gsem = (g0, g1)
    ssem = (s0, s1)

    def init_row(i, _):
      for q in range(h // LANES):
        rows_v[0, i, pl.ds(q * LANES, LANES)] = jnp.zeros((LANES,), jnp.float32)
      return _
    lax.fori_loop(0, B, init_row, None)

    @pl.loop(s, n_chunks, step=NS)
    def zero_chunk(kk):
      pltpu.sync_copy(rows_v.at[0, pl.ds(0, ZB)], acc_sh.at[pl.ds(kk * ZB, ZB)])

    @pl.loop(s, cnt_nchunks, step=NS)
    def zero_cnt(kk):
      pltpu.sync_copy(rows_v.at[0, pl.ds(0, cnt_rpt)],
                      cnt_sh.at[pl.ds(kk * cnt_rpt, cnt_rpt)])
    plsc.subcore_barrier()

    def split_src(b):
      for q in range(B // LANES):
        s16 = src_v[b, pl.ds(q * LANES, LANES)]
        hi_v[b, pl.ds(q * LANES, LANES)] = lax.shift_right_logical(s16, 7)
        lo_v[b, pl.ds(q * LANES, LANES)] = lax.bitwise_and(s16, CW - 1)

    def load_idx(j, b):
      pltpu.sync_copy(src_hbm.at[wid, j], src_v.at[b])
      pltpu.sync_copy(dst_hbm.at[wid, j], dst_v.at[b])

    load_idx(0, 0)
    pltpu.async_copy(xc_hbm.at[dst_v.at[0]], rows_v.at[0], gsem[0])
    split_src(0)
    pltpu.async_copy(oh_hbm.at[lo_v.at[0]], pay_v, gp)

    @pl.loop(0, nb // 2)
    def lp(j2):
      for b in (0, 1):
        b1 = 1 - b
        j = j2 * 2 + b

        @pl.when(j + 1 < nb)
        def _():
          @pl.when(j >= 1)
          def _():
            pltpu.make_async_copy(rows_v.at[b1], acc_sh.at[src_v.at[b1]],
                                  ssem[b1]).wait()
          load_idx(j + 1, b1)
          pltpu.async_copy(xc_hbm.at[dst_v.at[b1]], rows_v.at[b1], gsem[b1])
          split_src(b1)

        pltpu.make_async_copy(xc_hbm.at[dst_v.at[b]], rows_v.at[b], gsem[b]).wait()
        pltpu.async_copy(rows_v.at[b], acc_sh.at[src_v.at[b]], ssem[b], add=True)
        pltpu.make_async_copy(oh_hbm.at[lo_v.at[b]], pay_v, gp).wait()
        pltpu.sync_copy(pay_v, cnt_sh.at[hi_v.at[b]], add=True)

        @pl.when(j + 1 < nb)
        def _():
          pltpu.async_copy(oh_hbm.at[lo_v.at[b1]], pay_v, gp)

    for b in (0, 1):
      pltpu.make_async_copy(rows_v.at[b], acc_sh.at[src_v.at[b]], ssem[b]).wait()
    plsc.subcore_barrier()

    @pl.loop(s, n_chunks, step=NS)
    def out_chunk(kk):
      pltpu.sync_copy(acc_sh.at[pl.ds(kk * ZB, ZB)], rows_v.at[0, pl.ds(0, ZB)])
      pltpu.sync_copy(rows_v.at[0, pl.ds(0, ZB)], acc_out.at[c, pl.ds(kk * ZB, ZB)])

    @pl.loop(s, cnt_nchunks, step=NS)
    def out_cnt(kk):
      pltpu.sync_copy(cnt_sh.at[pl.ds(kk * cnt_rpt, cnt_rpt)], rows_v.at[1, pl.ds(0, cnt_rpt)])
      pltpu.sync_copy(rows_v.at[1, pl.ds(0, cnt_rpt)], cnt_out.at[c, pl.ds(kk * cnt_rpt, cnt_rpt)])

  return k


def _dense_body(xp_ref, acc_ref, cnt_ref, w1_ref, b1_ref, w2_ref, b2_ref,
                wu_ref, bu_ref, g_ref, bt_ref, out_ref):
  xp = xp_ref[...]
  ssum = acc_ref[0] + acc_ref[1]
  cnt = cnt_ref[0] + cnt_ref[1]
  hid = jnp.maximum(
      jnp.dot(xp, w1_ref[...], preferred_element_type=jnp.float32) + b1_ref[...],
      0.0)
  pred = jnp.dot(hid, w2_ref[...], preferred_element_type=jnp.float32) + b2_ref[...]
  agg = (ssum - cnt * pred) / jnp.maximum(cnt, 1.0)
  upd = jnp.dot(agg, wu_ref[...], preferred_element_type=jnp.float32) + bu_ref[...]
  t = xp + upd
  m = jnp.mean(t, axis=1, keepdims=True)
  v = jnp.mean((t - m) * (t - m), axis=1, keepdims=True)
  out_ref[...] = (t - m) * lax.rsqrt(v + 1e-5) * g_ref[...] + bt_ref[...]


def kernel(x_parent, x_child, edge_index, W1, b1, W2, b2, Wu, bu, gamma, beta):
  np_, h = x_parent.shape
  e = edge_index.shape[1]

  np_pad = -(-(np_ + 1) // 64) * 64
  chunk = NW * B
  e_pad = -(-e // (2 * chunk)) * (2 * chunk)
  nb = e_pad // chunk

  onehot = jnp.eye(CW, dtype=jnp.float32)

  src = edge_index[0]
  dst = edge_index[1]
  pad = e_pad - e
  if pad:
    src = jnp.concatenate([src, jnp.full((pad,), np_, jnp.int32)])
    dst = jnp.concatenate([dst, jnp.zeros((pad,), jnp.int32)])
  src3 = src.reshape(NW, nb, B)
  dst3 = dst.reshape(NW, nb, B)

  acc, cnt = _sc_segment_sum(nb, np_pad, h)(x_child, onehot, src3, dst3)
  cnt_col = cnt.reshape(NC, -1, 1)

  r = 1000 if np_ % 1000 == 0 else np_
  grid = (np_ // r,)
  new_parent = pl.pallas_call(
      _dense_body,
      grid=grid,
      in_specs=[
          pl.BlockSpec((r, h), lambda i: (i, 0)),
          pl.BlockSpec((NC, r, h), lambda i: (0, i, 0)),
          pl.BlockSpec((NC, r, 1), lambda i: (0, i, 0)),
          pl.BlockSpec((h, h), lambda i: (0, 0)),
          pl.BlockSpec((1, h), lambda i: (0, 0)),
          pl.BlockSpec((h, h), lambda i: (0, 0)),
          pl.BlockSpec((1, h), lambda i: (0, 0)),
          pl.BlockSpec((h, h), lambda i: (0, 0)),
          pl.BlockSpec((1, h), lambda i: (0, 0)),
          pl.BlockSpec((1, h), lambda i: (0, 0)),
          pl.BlockSpec((1, h), lambda i: (0, 0)),
      ],
      out_specs=pl.BlockSpec((r, h), lambda i: (i, 0)),
      out_shape=jax.ShapeDtypeStruct((np_, h), jnp.float32),
  )(x_parent, acc[:, :np_], cnt_col[:, :np_],
    W1, b1.reshape(1, h), W2, b2.reshape(1, h), Wu, bu.reshape(1, h),
    gamma.reshape(1, h), beta.reshape(1, h))

  return (new_parent, x_child)
